# baseline (speedup 1.0000x reference)
import functools

import numpy as np
import jax
import jax.numpy as jnp
from jax import lax
from jax.experimental import pallas as pl
from jax.experimental.pallas import tpu as pltpu
from jax.experimental.pallas import tpu_sc as plsc

N = 10000
E = 320000
D = 128
H = 128
G = 64
BNS = float(1.0 / np.sqrt(1.0 + 1e-5))

N_PAD = 10240
NC = 2
NT = 16
CH = 128
NCH = 81
EPW = NCH * CH
E_PAD = NC * NT * EPW
RPT = N_PAD // NT

RB = 1024
GRID = N_PAD // RB


def _sc_mesh():
    return plsc.VectorSubcoreMesh(core_axis_name="c", subcore_axis_name="s")


@functools.partial(
    pl.kernel,
    mesh=_sc_mesh(),
    out_type=jax.ShapeDtypeStruct((NC, N_PAD), jnp.float32),
    scratch_types=[
        pltpu.VMEM((NCH, CH), jnp.int32),
        pltpu.VMEM((CH,), jnp.float32),
        pltpu.VMEM((RPT,), jnp.float32),
        pltpu.VMEM_SHARED((N_PAD,), jnp.float32),
        pltpu.SemaphoreType.DMA,
    ],
)
def _sc_hist(dst_hbm, ones_hbm, zeros_hbm, out_hbm, idx_da, ones_v, zbuf,
             hacc, sem):
    c = lax.axis_index("c")
    s = lax.axis_index("s")
    wid = c * NT + s
    pltpu.sync_copy(dst_hbm.at[wid], idx_da)
    pltpu.sync_copy(zeros_hbm, zbuf)
    pltpu.sync_copy(zbuf, hacc.at[pl.ds(s * RPT, RPT)])
    pltpu.sync_copy(ones_hbm, ones_v)
    plsc.subcore_barrier()

    def fire(j, carry):
        pltpu.async_copy(ones_v, hacc.at[idx_da.at[j]], sem, add=True)
        return carry

    def drain(j, carry):
        pltpu.make_async_copy(ones_v, hacc.at[idx_da.at[j]], sem).wait()
        return carry

    lax.fori_loop(0, NCH, fire, 0)
    lax.fori_loop(0, NCH, drain, 0)
    plsc.subcore_barrier()
    pltpu.sync_copy(hacc.at[pl.ds(s * RPT, RPT)], zbuf)
    pltpu.sync_copy(zbuf, out_hbm.at[c, pl.ds(s * RPT, RPT)])


@functools.partial(
    pl.kernel,
    mesh=_sc_mesh(),
    out_type=jax.ShapeDtypeStruct((NC, N_PAD, H), jnp.float32),
    scratch_types=[
        pltpu.VMEM((NCH, CH), jnp.int32),
        [pltpu.VMEM((CH,), jnp.int32) for _ in range(2)],
        [pltpu.VMEM((CH, H), jnp.float32) for _ in range(2)],
        pltpu.VMEM_SHARED((N_PAD, H), jnp.float32),
        pltpu.SemaphoreType.DMA,
        pltpu.SemaphoreType.DMA,
        pltpu.SemaphoreType.DMA,
    ],
)
def _sc_spmm(h_hbm, src_hbm, dst_hbm, zeros_hbm, out_hbm,
             idx_sa, dbufs, rows, acc, gsem, dsem, ssem):
    c = lax.axis_index("c")
    s = lax.axis_index("s")
    wid = c * NT + s
    pltpu.sync_copy(src_hbm.at[wid], idx_sa)
    for k in range(RPT // CH):
        pltpu.async_copy(zeros_hbm, acc.at[pl.ds(s * RPT + k * CH, CH)], gsem)
    for k in range(RPT // CH):
        pltpu.make_async_copy(zeros_hbm,
                              acc.at[pl.ds(s * RPT + k * CH, CH)], gsem).wait()
    plsc.subcore_barrier()

    def gfire(j, b):
        pltpu.async_copy(dst_hbm.at[pl.ds(wid * EPW + j * CH, CH)],
                         dbufs[b], dsem)
        pltpu.async_copy(h_hbm.at[idx_sa.at[j]], rows[b], gsem)

    def gdrain(j, b):
        pltpu.make_async_copy(dst_hbm.at[pl.ds(wid * EPW + j * CH, CH)],
                              dbufs[b], dsem).wait()
        pltpu.make_async_copy(h_hbm.at[idx_sa.at[j]], rows[b], gsem).wait()

    def sfire(b):
        pltpu.async_copy(rows[b], acc.at[dbufs[b]], ssem, add=True)

    def sdrain(b):
        pltpu.make_async_copy(rows[b], acc.at[dbufs[b]], ssem).wait()

    gfire(0, 0)
    gdrain(0, 0)
    gfire(1, 1)
    sfire(0)

    def pair(jj, carry):
        j = 1 + 2 * jj
        gdrain(j, 1)
        sdrain(0)
        gfire(j + 1, 0)
        sfire(1)
        gdrain(j + 1, 0)
        sdrain(1)
        gfire(j + 2, 1)
        sfire(0)
        return carry

    lax.fori_loop(0, (NCH - 3) // 2, pair, 0)
    j = NCH - 2
    gdrain(j, 1)
    sdrain(0)
    gfire(j + 1, 0)
    sfire(1)
    gdrain(j + 1, 0)
    sdrain(1)
    sfire(0)
    sdrain(0)

    plsc.subcore_barrier()
    for k in range(RPT // CH):
        r0 = s * RPT + k * CH
        pltpu.sync_copy(acc.at[pl.ds(r0, CH)], rows[0])
        pltpu.sync_copy(rows[0], out_hbm.at[c, pl.ds(r0, CH)])


def _dinv(h0, h1):
    deg = h0 + h1 + 1.0
    return lax.rsqrt(jnp.maximum(deg, 1.0))


def _enc_body(x_ref, h0_ref, h1_ref, we_ref, be_ref, wc1_ref, out_ref):
    dinv = _dinv(h0_ref[...], h1_ref[...])
    henc = jnp.dot(x_ref[...], we_ref[...],
                   preferred_element_type=jnp.float32) + be_ref[...]
    out_ref[...] = jnp.dot(henc, wc1_ref[...],
                           preferred_element_type=jnp.float32) * dinv


def _mid_body(a0_ref, a1_ref, hp_ref, h0_ref, h1_ref, bc_ref, g_ref, be_ref,
              wn_ref, out_ref):
    dinv = _dinv(h0_ref[...], h1_ref[...])
    ssum = a0_ref[...] + a1_ref[...] + hp_ref[...]
    conv = ssum * dinv + bc_ref[...]
    hbn = conv * (g_ref[...] * BNS) + be_ref[...]
    hr = jnp.maximum(hbn, 0.0)
    out_ref[...] = jnp.dot(hr, wn_ref[...],
                           preferred_element_type=jnp.float32) * dinv


def _fin_body(a0_ref, a1_ref, hp_ref, h0_ref, h1_ref, bc_ref, g_ref, be_ref,
              bt_ref, wl1_ref, bl1_ref, wl2_ref, bl2_ref, out_ref,
              pool_ref, cnt_ref):
    i = pl.program_id(0)
    dinv = _dinv(h0_ref[...], h1_ref[...])
    ssum = a0_ref[...] + a1_ref[...] + hp_ref[...]
    conv = ssum * dinv + bc_ref[...]
    hbn = conv * (g_ref[...] * BNS) + be_ref[...]
    hr = jnp.maximum(hbn, 0.0)

    gid = lax.broadcasted_iota(jnp.int32, (RB, G), 1)
    oh = (bt_ref[...] == gid).astype(jnp.float32)
    pp = lax.dot_general(oh, hr, (((0,), (0,)), ((), ())),
                         preferred_element_type=jnp.float32)
    cc = lax.dot_general(oh, jnp.ones((RB, H), jnp.float32),
                         (((0,), (0,)), ((), ())),
                         preferred_element_type=jnp.float32)

    @pl.when(i == 0)
    def _():
        pool_ref[...] = jnp.zeros_like(pool_ref)
        cnt_ref[...] = jnp.zeros_like(cnt_ref)

    pool_ref[...] += pp
    cnt_ref[...] += cc

    @pl.when(i == GRID - 1)
    def _():
        pooled = pool_ref[...] / jnp.maximum(cnt_ref[...], 1.0)
        z = jnp.maximum(jnp.dot(pooled, wl1_ref[...],
                                preferred_element_type=jnp.float32)
                        + bl1_ref[...], 0.0)
        o = jnp.dot(z, wl2_ref[...],
                    preferred_element_type=jnp.float32) + bl2_ref[...]
        out_ref[...] = 1.0 / (1.0 + jnp.exp(-o))


def _row_spec(width):
    return pl.BlockSpec((RB, width), lambda i: (i, 0))


def _full_spec(r, c):
    return pl.BlockSpec((r, c), lambda i: (0, 0))


def _enc_call(x_p, h0, h1, W_enc, b_enc2, W_c1):
    return pl.pallas_call(
        _enc_body,
        grid=(GRID,),
        in_specs=[_row_spec(D), _row_spec(1), _row_spec(1),
                  _full_spec(D, H), _full_spec(1, H), _full_spec(H, H)],
        out_specs=_row_spec(H),
        out_shape=jax.ShapeDtypeStruct((N_PAD, H), jnp.float32),
    )(x_p, h0, h1, W_enc, b_enc2, W_c1)


def _mid_call(a0, a1, hp, h0, h1, bc, g, be, wn):
    return pl.pallas_call(
        _mid_body,
        grid=(GRID,),
        in_specs=[_row_spec(H), _row_spec(H), _row_spec(H),
                  _row_spec(1), _row_spec(1),
                  _full_spec(1, H), _full_spec(1, H), _full_spec(1, H),
                  _full_spec(H, H)],
        out_specs=_row_spec(H),
        out_shape=jax.ShapeDtypeStruct((N_PAD, H), jnp.float32),
    )(a0, a1, hp, h0, h1, bc, g, be, wn)


def _fin_call(a0, a1, hp, h0, h1, bc, g, be, bt, wl1, bl1, wl2, bl2):
    return pl.pallas_call(
        _fin_body,
        grid=(GRID,),
        in_specs=[_row_spec(H), _row_spec(H), _row_spec(H),
                  _row_spec(1), _row_spec(1),
                  _full_spec(1, H), _full_spec(1, H), _full_spec(1, H),
                  _row_spec(1),
                  _full_spec(H, G), _full_spec(1, G),
                  _full_spec(G, 1), _full_spec(1, 1)],
        out_specs=_full_spec(G, 1),
        out_shape=jax.ShapeDtypeStruct((G, 1), jnp.float32),
        scratch_shapes=[pltpu.VMEM((G, H), jnp.float32),
                        pltpu.VMEM((G, H), jnp.float32)],
    )(a0, a1, hp, h0, h1, bc, g, be, bt, wl1, bl1, wl2, bl2)


def kernel(x, edge_index, batch, W_enc, b_enc, W_c1, b_c1, g1, be1,
           W_c2, b_c2, g2, be2, W_l1, b_l1, W_l2, b_l2):
    src = edge_index[0]
    dst = edge_index[1]
    pad_e = E_PAD - E
    pad_rows = N + (jnp.arange(pad_e, dtype=jnp.int32) % (N_PAD - N))
    src_p = jnp.concatenate([src, pad_rows]).reshape(NC * NT, NCH, CH)
    dst_flat = jnp.concatenate([dst, pad_rows])
    dst_p = dst_flat.reshape(NC * NT, NCH, CH)
    x_p = jnp.pad(x, ((0, N_PAD - N), (0, 0)))
    batch_p = jnp.concatenate(
        [batch, jnp.full((N_PAD - N,), G, jnp.int32)]).reshape(N_PAD, 1)

    ones_ch = jnp.ones((CH,), jnp.float32)
    zeros_r = jnp.zeros((RPT,), jnp.float32)
    zeros_b = jnp.zeros((CH, H), jnp.float32)

    hist = _sc_hist(dst_p, ones_ch, zeros_r)
    h0 = hist[0].reshape(N_PAD, 1)
    h1 = hist[1].reshape(N_PAD, 1)

    h1p = _enc_call(x_p, h0, h1, W_enc, b_enc.reshape(1, H), W_c1)
    acc1 = _sc_spmm(h1p, src_p, dst_flat, zeros_b)
    h2p = _mid_call(acc1[0], acc1[1], h1p, h0, h1,
                    b_c1.reshape(1, H), g1.reshape(1, H), be1.reshape(1, H),
                    W_c2)
    acc2 = _sc_spmm(h2p, src_p, dst_flat, zeros_b)
    out = _fin_call(acc2[0], acc2[1], h2p, h0, h1,
                    b_c2.reshape(1, H), g2.reshape(1, H), be2.reshape(1, H),
                    batch_p, W_l1, b_l1.reshape(1, G), W_l2,
                    b_l2.reshape(1, 1))
    return out.reshape(G)


# speedup vs baseline: 1.0373x; 1.0373x over previous
import functools

import numpy as np
import jax
import jax.numpy as jnp
from jax import lax
from jax.experimental import pallas as pl
from jax.experimental.pallas import tpu as pltpu
from jax.experimental.pallas import tpu_sc as plsc

N = 10000
E = 320000
D = 128
H = 128
G = 64
BNS = float(1.0 / np.sqrt(1.0 + 1e-5))

N_PAD = 10240
NC = 2
NT = 16
CH = 128
NCH = 81
EPW = NCH * CH
E_PAD = NC * NT * EPW
RPT = N_PAD // NT

RB = 1024
GRID = N_PAD // RB


def _sc_mesh():
    return plsc.VectorSubcoreMesh(core_axis_name="c", subcore_axis_name="s")


@functools.partial(
    pl.kernel,
    mesh=_sc_mesh(),
    out_type=jax.ShapeDtypeStruct((NC, N_PAD), jnp.float32),
    scratch_types=[
        pltpu.VMEM((NCH, CH), jnp.int32),
        pltpu.VMEM((CH,), jnp.float32),
        pltpu.VMEM((RPT,), jnp.float32),
        pltpu.VMEM_SHARED((N_PAD,), jnp.float32),
        pltpu.SemaphoreType.DMA,
    ],
)
def _sc_hist(dst_hbm, ones_hbm, zeros_hbm, out_hbm, idx_da, ones_v, zbuf,
             hacc, sem):
    c = lax.axis_index("c")
    s = lax.axis_index("s")
    wid = c * NT + s
    pltpu.sync_copy(dst_hbm.at[wid], idx_da)
    pltpu.sync_copy(zeros_hbm, zbuf)
    pltpu.sync_copy(zbuf, hacc.at[pl.ds(s * RPT, RPT)])
    pltpu.sync_copy(ones_hbm, ones_v)
    plsc.subcore_barrier()

    def fire(j, carry):
        pltpu.async_copy(ones_v, hacc.at[idx_da.at[j]], sem, add=True)
        return carry

    def drain(j, carry):
        pltpu.make_async_copy(ones_v, hacc.at[idx_da.at[j]], sem).wait()
        return carry

    lax.fori_loop(0, NCH, fire, 0)
    lax.fori_loop(0, NCH, drain, 0)
    plsc.subcore_barrier()
    pltpu.sync_copy(hacc.at[pl.ds(s * RPT, RPT)], zbuf)
    pltpu.sync_copy(zbuf, out_hbm.at[c, pl.ds(s * RPT, RPT)])


@functools.partial(
    pl.kernel,
    mesh=_sc_mesh(),
    out_type=jax.ShapeDtypeStruct((NC, N_PAD, H), jnp.float32),
    scratch_types=[
        pltpu.VMEM((NCH, CH), jnp.int32),
        [pltpu.VMEM((CH,), jnp.int32) for _ in range(2)],
        [pltpu.VMEM((CH, H), jnp.float32) for _ in range(2)],
        pltpu.VMEM_SHARED((N_PAD, H), jnp.float32),
        pltpu.SemaphoreType.DMA,
        pltpu.SemaphoreType.DMA,
        pltpu.SemaphoreType.DMA,
    ],
)
def _sc_spmm(h_hbm, src_hbm, dst_hbm, zeros_hbm, out_hbm,
             idx_sa, dbufs, rows, acc, gsem, dsem, ssem):
    c = lax.axis_index("c")
    s = lax.axis_index("s")
    wid = c * NT + s
    pltpu.sync_copy(src_hbm.at[wid], idx_sa)
    for k in range(RPT // CH):
        pltpu.async_copy(zeros_hbm, acc.at[pl.ds(s * RPT + k * CH, CH)], gsem)
    for k in range(RPT // CH):
        pltpu.make_async_copy(zeros_hbm,
                              acc.at[pl.ds(s * RPT + k * CH, CH)], gsem).wait()
    plsc.subcore_barrier()

    def gfire(j, b):
        pltpu.async_copy(dst_hbm.at[pl.ds(wid * EPW + j * CH, CH)],
                         dbufs[b], dsem)
        pltpu.async_copy(h_hbm.at[idx_sa.at[j]], rows[b], gsem)

    def gdrain(j, b):
        pltpu.make_async_copy(dst_hbm.at[pl.ds(wid * EPW + j * CH, CH)],
                              dbufs[b], dsem).wait()
        pltpu.make_async_copy(h_hbm.at[idx_sa.at[j]], rows[b], gsem).wait()

    def sfire(b):
        pltpu.async_copy(rows[b], acc.at[dbufs[b]], ssem, add=True)

    def sdrain(b):
        pltpu.make_async_copy(rows[b], acc.at[dbufs[b]], ssem).wait()

    gfire(0, 0)
    gdrain(0, 0)
    gfire(1, 1)
    sfire(0)

    def pair(jj, carry):
        j = 1 + 2 * jj
        gdrain(j, 1)
        sdrain(0)
        gfire(j + 1, 0)
        sfire(1)
        gdrain(j + 1, 0)
        sdrain(1)
        gfire(j + 2, 1)
        sfire(0)
        return carry

    lax.fori_loop(0, (NCH - 3) // 2, pair, 0)
    j = NCH - 2
    gdrain(j, 1)
    sdrain(0)
    gfire(j + 1, 0)
    sfire(1)
    gdrain(j + 1, 0)
    sdrain(1)
    sfire(0)
    sdrain(0)

    plsc.subcore_barrier()
    for k in range(RPT // CH):
        r0 = s * RPT + k * CH
        pltpu.sync_copy(acc.at[pl.ds(r0, CH)], rows[0])
        pltpu.sync_copy(rows[0], out_hbm.at[c, pl.ds(r0, CH)])


def _dinv(h0, h1):
    deg = h0 + h1 + 1.0
    return lax.rsqrt(jnp.maximum(deg, 1.0))


def _enc_body(x_ref, h0_ref, h1_ref, we_ref, be_ref, wc1_ref, out_ref):
    dinv = _dinv(h0_ref[...], h1_ref[...])
    henc = jnp.dot(x_ref[...], we_ref[...],
                   preferred_element_type=jnp.float32) + be_ref[...]
    out_ref[...] = jnp.dot(henc, wc1_ref[...],
                           preferred_element_type=jnp.float32) * dinv


def _mid_body(a0_ref, a1_ref, hp_ref, h0_ref, h1_ref, bc_ref, g_ref, be_ref,
              wn_ref, out_ref):
    dinv = _dinv(h0_ref[...], h1_ref[...])
    ssum = a0_ref[0] + a1_ref[0] + hp_ref[...]
    conv = ssum * dinv + bc_ref[...]
    hbn = conv * (g_ref[...] * BNS) + be_ref[...]
    hr = jnp.maximum(hbn, 0.0)
    out_ref[...] = jnp.dot(hr, wn_ref[...],
                           preferred_element_type=jnp.float32) * dinv


def _fin_body(a0_ref, a1_ref, hp_ref, h0_ref, h1_ref, bc_ref, g_ref, be_ref,
              bt_ref, wl1_ref, bl1_ref, wl2_ref, bl2_ref, out_ref,
              pool_ref, cnt_ref):
    i = pl.program_id(0)
    dinv = _dinv(h0_ref[...], h1_ref[...])
    ssum = a0_ref[0] + a1_ref[0] + hp_ref[...]
    conv = ssum * dinv + bc_ref[...]
    hbn = conv * (g_ref[...] * BNS) + be_ref[...]
    hr = jnp.maximum(hbn, 0.0)

    gid = lax.broadcasted_iota(jnp.int32, (RB, G), 1)
    oh = (bt_ref[...] == gid).astype(jnp.float32)
    pp = lax.dot_general(oh, hr, (((0,), (0,)), ((), ())),
                         preferred_element_type=jnp.float32)
    cc = lax.dot_general(oh, jnp.ones((RB, H), jnp.float32),
                         (((0,), (0,)), ((), ())),
                         preferred_element_type=jnp.float32)

    @pl.when(i == 0)
    def _():
        pool_ref[...] = jnp.zeros_like(pool_ref)
        cnt_ref[...] = jnp.zeros_like(cnt_ref)

    pool_ref[...] += pp
    cnt_ref[...] += cc

    @pl.when(i == GRID - 1)
    def _():
        pooled = pool_ref[...] / jnp.maximum(cnt_ref[...], 1.0)
        z = jnp.maximum(jnp.dot(pooled, wl1_ref[...],
                                preferred_element_type=jnp.float32)
                        + bl1_ref[...], 0.0)
        o = jnp.dot(z, wl2_ref[...],
                    preferred_element_type=jnp.float32) + bl2_ref[...]
        out_ref[...] = 1.0 / (1.0 + jnp.exp(-o))


def _row_spec(width):
    return pl.BlockSpec((RB, width), lambda i: (i, 0))


def _full_spec(r, c):
    return pl.BlockSpec((r, c), lambda i: (0, 0))


def _enc_call(x_p, h0, h1, W_enc, b_enc2, W_c1):
    return pl.pallas_call(
        _enc_body,
        grid=(GRID,),
        in_specs=[_row_spec(D), _row_spec(1), _row_spec(1),
                  _full_spec(D, H), _full_spec(1, H), _full_spec(H, H)],
        out_specs=_row_spec(H),
        out_shape=jax.ShapeDtypeStruct((N_PAD, H), jnp.float32),
    )(x_p, h0, h1, W_enc, b_enc2, W_c1)


def _acc0_spec():
    return pl.BlockSpec((1, RB, H), lambda i: (0, i, 0))


def _acc1_spec():
    return pl.BlockSpec((1, RB, H), lambda i: (1, i, 0))


def _mid_call(acc, hp, h0, h1, bc, g, be, wn):
    return pl.pallas_call(
        _mid_body,
        grid=(GRID,),
        in_specs=[_acc0_spec(), _acc1_spec(), _row_spec(H),
                  _row_spec(1), _row_spec(1),
                  _full_spec(1, H), _full_spec(1, H), _full_spec(1, H),
                  _full_spec(H, H)],
        out_specs=_row_spec(H),
        out_shape=jax.ShapeDtypeStruct((N_PAD, H), jnp.float32),
    )(acc, acc, hp, h0, h1, bc, g, be, wn)


def _fin_call(acc, hp, h0, h1, bc, g, be, bt, wl1, bl1, wl2, bl2):
    return pl.pallas_call(
        _fin_body,
        grid=(GRID,),
        in_specs=[_acc0_spec(), _acc1_spec(), _row_spec(H),
                  _row_spec(1), _row_spec(1),
                  _full_spec(1, H), _full_spec(1, H), _full_spec(1, H),
                  _row_spec(1),
                  _full_spec(H, G), _full_spec(1, G),
                  _full_spec(G, 1), _full_spec(1, 1)],
        out_specs=_full_spec(G, 1),
        out_shape=jax.ShapeDtypeStruct((G, 1), jnp.float32),
        scratch_shapes=[pltpu.VMEM((G, H), jnp.float32),
                        pltpu.VMEM((G, H), jnp.float32)],
    )(acc, acc, hp, h0, h1, bc, g, be, bt, wl1, bl1, wl2, bl2)


def kernel(x, edge_index, batch, W_enc, b_enc, W_c1, b_c1, g1, be1,
           W_c2, b_c2, g2, be2, W_l1, b_l1, W_l2, b_l2):
    src = edge_index[0]
    dst = edge_index[1]
    pad_e = E_PAD - E
    pad_rows = N + (jnp.arange(pad_e, dtype=jnp.int32) % (N_PAD - N))
    src_p = jnp.concatenate([src, pad_rows]).reshape(NC * NT, NCH, CH)
    dst_flat = jnp.concatenate([dst, pad_rows])
    dst_p = dst_flat.reshape(NC * NT, NCH, CH)
    x_p = jnp.pad(x, ((0, N_PAD - N), (0, 0)))
    batch_p = jnp.concatenate(
        [batch, jnp.full((N_PAD - N,), G, jnp.int32)]).reshape(N_PAD, 1)

    ones_ch = jnp.ones((CH,), jnp.float32)
    zeros_r = jnp.zeros((RPT,), jnp.float32)
    zeros_b = jnp.zeros((CH, H), jnp.float32)

    hist = _sc_hist(dst_p, ones_ch, zeros_r)
    h0 = hist[0].reshape(N_PAD, 1)
    h1 = hist[1].reshape(N_PAD, 1)

    h1p = _enc_call(x_p, h0, h1, W_enc, b_enc.reshape(1, H), W_c1)
    acc1 = _sc_spmm(h1p, src_p, dst_flat, zeros_b)
    h2p = _mid_call(acc1, h1p, h0, h1,
                    b_c1.reshape(1, H), g1.reshape(1, H), be1.reshape(1, H),
                    W_c2)
    acc2 = _sc_spmm(h2p, src_p, dst_flat, zeros_b)
    out = _fin_call(acc2, h2p, h0, h1,
                    b_c2.reshape(1, H), g2.reshape(1, H), be2.reshape(1, H),
                    batch_p, W_l1, b_l1.reshape(1, G), W_l2,
                    b_l2.reshape(1, 1))
    return out.reshape(G)
